# Initial kernel scaffold; baseline (speedup 1.0000x reference)
import jax
import jax.numpy as jnp
from jax.experimental import pallas as pl


def kernel(input_ids, position_ids, word_table, pos_table):
    raise NotImplementedError("write your pallas kernel here")


# speedup vs baseline: 1.2616x; 1.2616x over previous
import functools

import jax
import jax.numpy as jnp
from jax import lax
from jax.experimental import pallas as pl
from jax.experimental.pallas import tpu as pltpu
from jax.experimental.pallas import tpu_sc as plsc

_HIDDEN = 1024
_LANES = 16
_NCORES = 2
_NSUB = 16
_NW = _NCORES * _NSUB

_CHUNK = 32


def _emb_body(ids_hbm, pids_hbm, word_hbm, ptab_hbm, out_hbm,
              idx_w, idx_p, buf_w, buf_p, sem_w, sem_p, *, per_w, nchunk):
    wid = lax.axis_index("s") * _NCORES + lax.axis_index("c")
    base = wid * per_w
    pltpu.sync_copy(ids_hbm.at[pl.ds(base, per_w)], idx_w)
    pltpu.sync_copy(pids_hbm.at[pl.ds(base, per_w)], idx_p)

    def chunk_body(ci, carry):
        off = ci * _CHUNK
        cw = pltpu.async_copy(word_hbm.at[idx_w.at[pl.ds(off, _CHUNK)]],
                              buf_w, sem_w)
        cp = pltpu.async_copy(ptab_hbm.at[idx_p.at[pl.ds(off, _CHUNK)]],
                              buf_p, sem_p)
        cw.wait()
        cp.wait()

        def add_row(r, c2):
            for j in range(_HIDDEN // _LANES):
                sl = pl.ds(j * _LANES, _LANES)
                plsc.addupdate(buf_w.at[r, sl], buf_p[r, sl])
            return c2

        lax.fori_loop(0, _CHUNK, add_row, 0, unroll=False)
        pltpu.sync_copy(buf_w, out_hbm.at[pl.ds(base + off, _CHUNK)])
        return carry

    lax.fori_loop(0, nchunk, chunk_body, 0, unroll=False)


def kernel(input_ids, position_ids, word_table, pos_table):
    b, s = input_ids.shape
    n = b * s
    per_w = n // _NW
    nchunk = per_w // _CHUNK
    ids = input_ids.reshape(n).astype(jnp.int32)
    pids = position_ids.reshape(n).astype(jnp.int32)

    mesh = plsc.VectorSubcoreMesh(core_axis_name="c", subcore_axis_name="s")
    grid_kernel = pl.kernel(
        functools.partial(_emb_body, per_w=per_w, nchunk=nchunk),
        mesh=mesh,
        out_type=jax.ShapeDtypeStruct((n, _HIDDEN), jnp.float32),
        scratch_types=[
            pltpu.VMEM((per_w,), jnp.int32),
            pltpu.VMEM((per_w,), jnp.int32),
            pltpu.VMEM((_CHUNK, _HIDDEN), jnp.float32),
            pltpu.VMEM((_CHUNK, _HIDDEN), jnp.float32),
            pltpu.SemaphoreType.DMA,
            pltpu.SemaphoreType.DMA,
        ],
    )
    out = grid_kernel(ids, pids, word_table, pos_table)
    return out.reshape(b, s, _HIDDEN)


# speedup vs baseline: 1.7651x; 1.3991x over previous
import functools

import jax
import jax.numpy as jnp
from jax import lax
from jax.experimental import pallas as pl
from jax.experimental.pallas import tpu as pltpu
from jax.experimental.pallas import tpu_sc as plsc

_HIDDEN = 1024
_LANES = 16
_NCORES = 2
_NSUB = 16
_NW = _NCORES * _NSUB

_CHUNK = 16


def _emb_body(ids_hbm, pids_hbm, word_hbm, ptab_hbm, out_hbm,
              idx_w, idx_p, w0, w1, p0, p1, o0, o1,
              sw0, sw1, sp0, sp1, so0, so1, *, per_w, nchunk):
    wid = lax.axis_index("s") * _NCORES + lax.axis_index("c")
    base = wid * per_w
    w = (w0, w1)
    p = (p0, p1)
    o = (o0, o1)
    sw = (sw0, sw1)
    sp = (sp0, sp1)
    so = (so0, so1)

    pltpu.sync_copy(ids_hbm.at[pl.ds(base, per_w)], idx_w)
    pltpu.sync_copy(pids_hbm.at[pl.ds(base, per_w)], idx_p)

    def gather_pair(ci, b):
        off = ci * _CHUNK
        pltpu.make_async_copy(word_hbm.at[idx_w.at[pl.ds(off, _CHUNK)]],
                              w[b], sw[b]).start()
        pltpu.make_async_copy(ptab_hbm.at[idx_p.at[pl.ds(off, _CHUNK)]],
                              p[b], sp[b]).start()

    def wait_gather(ci, b):
        off = ci * _CHUNK
        pltpu.make_async_copy(word_hbm.at[idx_w.at[pl.ds(off, _CHUNK)]],
                              w[b], sw[b]).wait()
        pltpu.make_async_copy(ptab_hbm.at[idx_p.at[pl.ds(off, _CHUNK)]],
                              p[b], sp[b]).wait()

    def start_out(ci, b):
        pltpu.make_async_copy(o[b], out_hbm.at[pl.ds(base + ci * _CHUNK,
                                                     _CHUNK)], so[b]).start()

    def wait_out(ci, b):
        pltpu.make_async_copy(o[b], out_hbm.at[pl.ds(base + ci * _CHUNK,
                                                     _CHUNK)], so[b]).wait()

    gather_pair(0, 0)

    def pair_body(g, carry):
        for b in (0, 1):
            ci = 2 * g + b
            if b == 0:
                gather_pair(ci + 1, 1 - b)
            else:
                @pl.when(g < (nchunk // 2) - 1)
                def _():
                    gather_pair(ci + 1, 1 - b)
            wait_gather(ci, b)

            @pl.when(g > 0)
            def _():
                wait_out(ci - 2, b)

            def add_row(r, c2):
                for j in range(_HIDDEN // _LANES):
                    sl = pl.ds(j * _LANES, _LANES)
                    o[b][r, sl] = w[b][r, sl] + p[b][r, sl]
                return c2

            lax.fori_loop(0, _CHUNK, add_row, 0, unroll=False)
            start_out(ci, b)
        return carry

    lax.fori_loop(0, nchunk // 2, pair_body, 0, unroll=False)
    wait_out(nchunk - 2, 0)
    wait_out(nchunk - 1, 1)


def kernel(input_ids, position_ids, word_table, pos_table):
    b, s = input_ids.shape
    n = b * s
    per_w = n // _NW
    nchunk = per_w // _CHUNK
    ids = input_ids.reshape(n).astype(jnp.int32)
    pids = position_ids.reshape(n).astype(jnp.int32)

    mesh = plsc.VectorSubcoreMesh(core_axis_name="c", subcore_axis_name="s")
    grid_kernel = pl.kernel(
        functools.partial(_emb_body, per_w=per_w, nchunk=nchunk),
        mesh=mesh,
        out_type=jax.ShapeDtypeStruct((n, _HIDDEN), jnp.float32),
        scratch_types=[
            pltpu.VMEM((per_w,), jnp.int32),
            pltpu.VMEM((per_w,), jnp.int32),
            pltpu.VMEM((_CHUNK, _HIDDEN), jnp.float32),
            pltpu.VMEM((_CHUNK, _HIDDEN), jnp.float32),
            pltpu.VMEM((_CHUNK, _HIDDEN), jnp.float32),
            pltpu.VMEM((_CHUNK, _HIDDEN), jnp.float32),
            pltpu.VMEM((_CHUNK, _HIDDEN), jnp.float32),
            pltpu.VMEM((_CHUNK, _HIDDEN), jnp.float32),
            pltpu.SemaphoreType.DMA,
            pltpu.SemaphoreType.DMA,
            pltpu.SemaphoreType.DMA,
            pltpu.SemaphoreType.DMA,
            pltpu.SemaphoreType.DMA,
            pltpu.SemaphoreType.DMA,
        ],
    )
    out = grid_kernel(ids, pids, word_table, pos_table)
    return out.reshape(b, s, _HIDDEN)
